# baseline (speedup 1.0000x reference)
import functools
import math

import jax
import jax.numpy as jnp
from jax import lax
from jax.experimental import pallas as pl
from jax.experimental.pallas import tpu as pltpu
from jax.experimental.pallas import tpu_sc as plsc

D = 64
SCALE = math.sqrt(D)
LANES = 16

NC = 2
NS = 16
NW = NC * NS

T_DIM = 50
B_DIM = 16384
BLK = 128
BPW = 4
N_BLOCKS = T_DIM * BPW
NBUF = 4


def _make_lookup():
    mesh = plsc.VectorSubcoreMesh(core_axis_name="c", subcore_axis_name="s")

    @functools.partial(
        pl.kernel,
        out_type=jax.ShapeDtypeStruct((T_DIM, 8, B_DIM // BLK, 8, BLK),
                                      jnp.float32),
        mesh=mesh,
        scratch_types=[
            pltpu.VMEM((NBUF, BLK), jnp.int32),
            pltpu.VMEM((NBUF, BLK), jnp.int32),
            pltpu.VMEM((NBUF * BLK, BLK), jnp.float32),
            pltpu.VMEM((2, D, BLK), jnp.float32),
            pltpu.SemaphoreType.DMA((NBUF,)),
            pltpu.SemaphoreType.DMA((NBUF,)),
            pltpu.SemaphoreType.DMA((2,)),
        ],
        compiler_params=pltpu.CompilerParams(needs_layout_passes=False),
    )
    def lookup(xt_hbm, w2_hbm, out_hbm, idx_v, q_v, rows_v, trans_v,
               isem, gsem, ssem):
        wid = lax.axis_index("s") * NC + lax.axis_index("c")
        bi0 = wid * BPW
        iota = lax.iota(jnp.int32, LANES)

        def fire_idx(t, bj, slot):
            pltpu.async_copy(
                xt_hbm.at[t, pl.ds((bi0 + bj) * BLK, BLK)],
                idx_v.at[slot], isem.at[slot])

        def wait_idx(slot):
            pltpu.make_async_copy(
                xt_hbm.at[0, pl.ds(0, BLK)], idx_v.at[slot],
                isem.at[slot]).wait()

        def fire_gather(buf, slot):
            for j in range(BLK // LANES):
                sl = pl.ds(j * LANES, LANES)
                q_v[buf, sl] = idx_v[slot, sl] >> 1
            pltpu.async_copy(
                w2_hbm.at[q_v.at[buf]],
                rows_v.at[pl.ds(buf * BLK, BLK)], gsem.at[buf])

        def wait_gather(buf):
            pltpu.make_async_copy(
                w2_hbm.at[pl.ds(0, BLK)],
                rows_v.at[pl.ds(buf * BLK, BLK)], gsem.at[buf]).wait()

        def transpose_scale(buf, half, slot):
            def br_body(br0, _):
                b0 = br0 * LANES
                row_vec = iota + (buf * BLK + b0)
                idxv = idx_v[slot, pl.ds(b0, LANES)]
                off_vec = (idxv & 1) * D

                @plsc.parallel_loop(0, D, step=1, unroll=16)
                def _(c):
                    v = plsc.load_gather(rows_v, [row_vec, off_vec + c])
                    trans_v[half, c, pl.ds(b0, LANES)] = v * SCALE

                return ()

            lax.fori_loop(0, BLK // LANES, br_body, ())

        def fire_store(t, bj, half):
            for ci in range(8):
                pltpu.async_copy(trans_v.at[half, pl.ds(ci * 8, 8)],
                                 out_hbm.at[t, ci, bi0 + bj], ssem.at[half])

        def wait_store(half):
            for ci in range(8):
                pltpu.make_async_copy(trans_v.at[half, pl.ds(ci * 8, 8)],
                                      out_hbm.at[0, ci, 0],
                                      ssem.at[half]).wait()

        def step(t, j, first=False, prefetch=True):
            half = j % 2
            j3 = j + 3
            if prefetch:
                fire_idx(t + j3 // BPW, j3 % BPW, j3 % NBUF)
            wait_gather(j)
            if not first:
                wait_store(half)
            transpose_scale(j, half, j)
            fire_store(t, j, half)
            if prefetch:
                wait_idx(j3 % NBUF)
                fire_gather(j3 % NBUF, j3 % NBUF)

        for k in range(NBUF - 1):
            fire_idx(0, k, k)
        for k in range(NBUF - 1):
            wait_idx(k)
            fire_gather(k, k)

        for j in range(BPW):
            step(0, j, first=(j < 2))

        def group(t, _):
            for j in range(BPW):
                step(t, j)
            return ()

        lax.fori_loop(1, T_DIM - 1, group, ())

        for j in range(BPW):
            step(T_DIM - 1, j, prefetch=(j < 1))

        wait_store(0)
        wait_store(1)

    return lookup


def kernel(x, W):
    xt = x.T.astype(jnp.int32)
    w2 = W.reshape(500000, 128)
    out5 = _make_lookup()(xt, w2)
    return out5.transpose(2, 4, 0, 1, 3).reshape(B_DIM, T_DIM, D)


# speedup vs baseline: 1.0050x; 1.0050x over previous
import functools
import math

import jax
import jax.numpy as jnp
from jax import lax
from jax.experimental import pallas as pl
from jax.experimental.pallas import tpu as pltpu
from jax.experimental.pallas import tpu_sc as plsc

D = 64
SCALE = math.sqrt(D)
LANES = 16

NC = 2
NS = 16
NW = NC * NS

T_DIM = 50
B_DIM = 16384
BLK = 128
BPW = 4
N_BLOCKS = T_DIM * BPW
NBUF = 4


def _make_lookup():
    mesh = plsc.VectorSubcoreMesh(core_axis_name="c", subcore_axis_name="s")

    @functools.partial(
        pl.kernel,
        out_type=jax.ShapeDtypeStruct((T_DIM, 8, B_DIM // BLK, 8, BLK),
                                      jnp.float32),
        mesh=mesh,
        scratch_types=[
            pltpu.VMEM((NBUF, BLK), jnp.int32),
            pltpu.VMEM((NBUF, BLK), jnp.int32),
            pltpu.VMEM((NBUF * BLK, BLK), jnp.float32),
            pltpu.VMEM((2, 8, 8, BLK), jnp.float32),
            pltpu.SemaphoreType.DMA((NBUF,)),
            pltpu.SemaphoreType.DMA((NBUF,)),
            pltpu.SemaphoreType.DMA((2,)),
        ],
        compiler_params=pltpu.CompilerParams(needs_layout_passes=False),
    )
    def lookup(xt_hbm, w2_hbm, out_hbm, idx_v, q_v, rows_v, trans_v,
               isem, gsem, ssem):
        wid = lax.axis_index("s") * NC + lax.axis_index("c")
        bi0 = wid * BPW
        iota = lax.iota(jnp.int32, LANES)

        def fire_idx(t, bj, slot):
            pltpu.async_copy(
                xt_hbm.at[t, pl.ds((bi0 + bj) * BLK, BLK)],
                idx_v.at[slot], isem.at[slot])

        def wait_idx(slot):
            pltpu.make_async_copy(
                xt_hbm.at[0, pl.ds(0, BLK)], idx_v.at[slot],
                isem.at[slot]).wait()

        def fire_gather(buf, slot):
            for j in range(BLK // LANES):
                sl = pl.ds(j * LANES, LANES)
                q_v[buf, sl] = idx_v[slot, sl] >> 1
            pltpu.async_copy(
                w2_hbm.at[q_v.at[buf]],
                rows_v.at[pl.ds(buf * BLK, BLK)], gsem.at[buf])

        def wait_gather(buf):
            pltpu.make_async_copy(
                w2_hbm.at[pl.ds(0, BLK)],
                rows_v.at[pl.ds(buf * BLK, BLK)], gsem.at[buf]).wait()

        def transpose_scale(buf, half, slot):
            def br_body(br0, _):
                b0 = br0 * LANES
                row_vec = iota + (buf * BLK + b0)
                idxv = idx_v[slot, pl.ds(b0, LANES)]
                off_vec = (idxv & 1) * D

                @plsc.parallel_loop(0, D, step=1, unroll=16)
                def _(c):
                    v = plsc.load_gather(rows_v, [row_vec, off_vec + c])
                    trans_v[half, c >> 3, c & 7, pl.ds(b0, LANES)] = v * SCALE

                return ()

            lax.fori_loop(0, BLK // LANES, br_body, ())

        def fire_store(t, bj, half):
            pltpu.async_copy(trans_v.at[half],
                             out_hbm.at[t, pl.ds(0, 8), bi0 + bj],
                             ssem.at[half])

        def wait_store(half):
            pltpu.make_async_copy(trans_v.at[half],
                                  out_hbm.at[0, pl.ds(0, 8), 0],
                                  ssem.at[half]).wait()

        def step(t, j, first=False, prefetch=True):
            half = j % 2
            j3 = j + 3
            if prefetch:
                fire_idx(t + j3 // BPW, j3 % BPW, j3 % NBUF)
            wait_gather(j)
            if not first:
                wait_store(half)
            transpose_scale(j, half, j)
            fire_store(t, j, half)
            if prefetch:
                wait_idx(j3 % NBUF)
                fire_gather(j3 % NBUF, j3 % NBUF)

        for k in range(NBUF - 1):
            fire_idx(0, k, k)
        for k in range(NBUF - 1):
            wait_idx(k)
            fire_gather(k, k)

        for j in range(BPW):
            step(0, j, first=(j < 2))

        def group(t, _):
            for j in range(BPW):
                step(t, j)
            return ()

        lax.fori_loop(1, T_DIM - 1, group, ())

        for j in range(BPW):
            step(T_DIM - 1, j, prefetch=(j < 1))

        wait_store(0)
        wait_store(1)

    return lookup


def kernel(x, W):
    xt = x.T.astype(jnp.int32)
    w2 = W.reshape(500000, 128)
    out5 = _make_lookup()(xt, w2)
    return out5.transpose(2, 4, 0, 1, 3).reshape(B_DIM, T_DIM, D)
